# Initial kernel scaffold; baseline (speedup 1.0000x reference)
import jax
import jax.numpy as jnp
from jax.experimental import pallas as pl


def kernel(input_ids, item_table, pos_table, ln_gamma, ln_beta):
    raise NotImplementedError("write your pallas kernel here")


# speedup vs baseline: 1.3120x; 1.3120x over previous
import functools

import jax
import jax.numpy as jnp
from jax import lax
from jax.experimental import pallas as pl
from jax.experimental.pallas import tpu as pltpu
from jax.experimental.pallas import tpu_sc as plsc

_BATCH = 4096
_SEQ = 200
_HID = 64
_ROWS = _BATCH * _SEQ
_NW = 32
_RPW = _ROWS // _NW
_CHUNK = 256
_NCH = _RPW // _CHUNK
_SUB = 128
_NSUB = _CHUNK // _SUB
_LANES = 16
_NV = _HID // _LANES
_EPS = 1e-12


def _hsum16(v):
    lanes = lax.iota(jnp.int32, _LANES)
    dnums = lax.GatherDimensionNumbers(
        offset_dims=(), collapsed_slice_dims=(0,), start_index_map=(0,))
    for k in (1, 2, 4, 8):
        idx = lax.bitwise_xor(lanes, jnp.int32(k))
        v = v + lax.gather(v, idx[:, None], dnums, slice_sizes=(1,),
                           mode=lax.GatherScatterMode.PROMISE_IN_BOUNDS)
    return v


def _rsqrt16(v):
    i = lax.bitcast_convert_type(v, jnp.int32)
    i = jnp.int32(0x5F3759DF) - lax.shift_right_arithmetic(i, jnp.int32(1))
    y = lax.bitcast_convert_type(i, jnp.float32)
    half = v * 0.5
    for _ in range(3):
        y = y * (1.5 - half * y * y)
    return y


def _sc_body(ids_hbm, table_hbm, pos_hbm, gam_hbm, bet_hbm, out_hbm,
             idx_v, rows_v, pos_v, gam_v, bet_v, sem):
    wid = lax.axis_index("s") * 2 + lax.axis_index("c")
    base = wid * _RPW

    pltpu.sync_copy(pos_hbm, pos_v)
    pltpu.sync_copy(gam_hbm, gam_v)
    pltpu.sync_copy(bet_hbm, bet_v)

    def chunk_body(g, carry):
        row0 = base + g * _CHUNK
        for j in range(_NSUB):
            pltpu.sync_copy(ids_hbm.at[pl.ds(row0 + j * _SUB, _SUB)],
                            idx_v.at[j])
        copies = []
        for j in range(_NSUB):
            copies.append(pltpu.async_copy(
                table_hbm.at[idx_v.at[j]],
                rows_v.at[pl.ds(j * _SUB, _SUB), :],
                sem))
        for cp in copies:
            cp.wait()

        tbase = (g * _CHUNK) % _SEQ

        def row_body(i, carry2):
            t = (tbase + i) % _SEQ
            x = [rows_v[i, pl.ds(c * _LANES, _LANES)] for c in range(_NV)]
            p = [pos_v[t, pl.ds(c * _LANES, _LANES)] for c in range(_NV)]
            x = [xc + pc for xc, pc in zip(x, p)]
            s = x[0] + x[1] + x[2] + x[3]
            q = x[0] * x[0] + x[1] * x[1] + x[2] * x[2] + x[3] * x[3]
            mu = _hsum16(s) * (1.0 / _HID)
            var = _hsum16(q) * (1.0 / _HID) - mu * mu
            rstd = _rsqrt16(var + _EPS)
            for c in range(_NV):
                gv = gam_v[pl.ds(c * _LANES, _LANES)]
                bv = bet_v[pl.ds(c * _LANES, _LANES)]
                rows_v[i, pl.ds(c * _LANES, _LANES)] = (
                    (x[c] - mu) * rstd * gv + bv)
            return carry2

        lax.fori_loop(0, _CHUNK, row_body, 0)

        pltpu.sync_copy(rows_v, out_hbm.at[pl.ds(row0, _CHUNK), :])
        return carry

    lax.fori_loop(0, _NCH, chunk_body, 0)


_mesh = plsc.VectorSubcoreMesh(core_axis_name="c", subcore_axis_name="s")

_sc_kernel = functools.partial(
    pl.kernel,
    out_type=jax.ShapeDtypeStruct((_ROWS, _HID), jnp.float32),
    mesh=_mesh,
    compiler_params=pltpu.CompilerParams(use_tc_tiling_on_sc=False),
    scratch_types=[
        pltpu.VMEM((_NSUB, _SUB), jnp.int32),
        pltpu.VMEM((_CHUNK, _HID), jnp.float32),
        pltpu.VMEM((_SEQ, _HID), jnp.float32),
        pltpu.VMEM((_HID,), jnp.float32),
        pltpu.VMEM((_HID,), jnp.float32),
        pltpu.SemaphoreType.DMA,
    ],
)(_sc_body)


@jax.jit
def kernel(input_ids, item_table, pos_table, ln_gamma, ln_beta):
    ids_flat = input_ids.reshape(_ROWS).astype(jnp.int32)
    out = _sc_kernel(ids_flat, item_table, pos_table, ln_gamma, ln_beta)
    return out.reshape(_BATCH, _SEQ, _HID)


# speedup vs baseline: 2.4115x; 1.8380x over previous
import functools

import jax
import jax.numpy as jnp
from jax import lax
from jax.experimental import pallas as pl
from jax.experimental.pallas import tpu as pltpu
from jax.experimental.pallas import tpu_sc as plsc

_BATCH = 4096
_SEQ = 200
_HID = 64
_ROWS = _BATCH * _SEQ
_NW = 32
_RPW = _ROWS // _NW
_CHUNK = _SEQ
_NCH = _RPW // _CHUNK
_NPAIR = _NCH // 2
_SUBS = ((0, 128), (128, 72))
_LANES = 16
_NV = _HID // _LANES
_EPS = 1e-12


def _hsum16(v):
    lanes = lax.iota(jnp.int32, _LANES)
    dnums = lax.GatherDimensionNumbers(
        offset_dims=(), collapsed_slice_dims=(0,), start_index_map=(0,))
    for k in (1, 2, 4, 8):
        idx = lax.bitwise_xor(lanes, jnp.int32(k))
        v = v + lax.gather(v, idx[:, None], dnums, slice_sizes=(1,),
                           mode=lax.GatherScatterMode.PROMISE_IN_BOUNDS)
    return v


def _rsqrt16(v):
    i = lax.bitcast_convert_type(v, jnp.int32)
    i = jnp.int32(0x5F3759DF) - lax.shift_right_arithmetic(i, jnp.int32(1))
    y = lax.bitcast_convert_type(i, jnp.float32)
    half = v * 0.5
    for _ in range(3):
        y = y * (1.5 - half * y * y)
    return y


def _sc_body(ids_hbm, table_hbm, pos_hbm, gam_hbm, bet_hbm, out_hbm,
             idx_all, rows0, rows1, pos_v, gam_v, bet_v, gsem, ssem):
    wid = lax.axis_index("s") * 2 + lax.axis_index("c")
    base = wid * _RPW

    pltpu.sync_copy(pos_hbm, pos_v)
    pltpu.sync_copy(gam_hbm, gam_v)
    pltpu.sync_copy(bet_hbm, bet_v)
    pltpu.sync_copy(ids_hbm.at[pl.ds(base, _RPW)], idx_all)

    gv = [gam_v[pl.ds(c * _LANES, _LANES)] for c in range(_NV)]
    bv = [bet_v[pl.ds(c * _LANES, _LANES)] for c in range(_NV)]

    def issue_gather(goff, buf):
        for off, n in _SUBS:
            pltpu.make_async_copy(
                table_hbm.at[idx_all.at[pl.ds(goff + off, n)]],
                buf.at[pl.ds(off, n), :],
                gsem).start()

    def wait_gather(buf):
        pltpu.make_async_copy(
            table_hbm.at[pl.ds(0, _CHUNK), :], buf, gsem).wait()

    def issue_store(buf, row0):
        pltpu.make_async_copy(
            buf, out_hbm.at[pl.ds(row0, _CHUNK), :], ssem).start()

    def wait_store():
        pltpu.make_async_copy(
            table_hbm.at[pl.ds(0, _CHUNK), :], rows0, ssem).wait()

    def compute(buf):
        @plsc.parallel_loop(0, _CHUNK, 1, unroll=8)
        def _row(i):
            x = [buf[i, pl.ds(c * _LANES, _LANES)]
                 + pos_v[i, pl.ds(c * _LANES, _LANES)] for c in range(_NV)]
            s = x[0] + x[1] + x[2] + x[3]
            q = x[0] * x[0] + x[1] * x[1] + x[2] * x[2] + x[3] * x[3]
            mu = _hsum16(s) * (1.0 / _HID)
            var = _hsum16(q) * (1.0 / _HID) - mu * mu
            rstd = _rsqrt16(var + _EPS)
            ms = mu * rstd
            for c in range(_NV):
                buf[i, pl.ds(c * _LANES, _LANES)] = (
                    (x[c] * rstd - ms) * gv[c] + bv[c])

    def pair(k, carry):
        a = 2 * k
        row0a = base + a * _CHUNK
        wait_gather(rows0)

        @pl.when(k >= 1)
        def _():
            wait_store()

        issue_gather((a + 1) * _CHUNK, rows1)
        compute(rows0)
        issue_store(rows0, row0a)

        wait_gather(rows1)
        wait_store()

        @pl.when(k < _NPAIR - 1)
        def _():
            issue_gather((a + 2) * _CHUNK, rows0)

        compute(rows1)
        issue_store(rows1, row0a + _CHUNK)
        return carry

    issue_gather(0, rows0)
    lax.fori_loop(0, _NPAIR, pair, 0)
    wait_store()


_mesh = plsc.VectorSubcoreMesh(core_axis_name="c", subcore_axis_name="s")

_sc_kernel = functools.partial(
    pl.kernel,
    out_type=jax.ShapeDtypeStruct((_ROWS, _HID), jnp.float32),
    mesh=_mesh,
    compiler_params=pltpu.CompilerParams(use_tc_tiling_on_sc=False),
    scratch_types=[
        pltpu.VMEM((_RPW,), jnp.int32),
        pltpu.VMEM((_CHUNK, _HID), jnp.float32),
        pltpu.VMEM((_CHUNK, _HID), jnp.float32),
        pltpu.VMEM((_SEQ, _HID), jnp.float32),
        pltpu.VMEM((_HID,), jnp.float32),
        pltpu.VMEM((_HID,), jnp.float32),
        pltpu.SemaphoreType.DMA,
        pltpu.SemaphoreType.DMA,
    ],
)(_sc_body)


@jax.jit
def kernel(input_ids, item_table, pos_table, ln_gamma, ln_beta):
    ids_flat = input_ids.reshape(_ROWS).astype(jnp.int32)
    out = _sc_kernel(ids_flat, item_table, pos_table, ln_gamma, ln_beta)
    return out.reshape(_BATCH, _SEQ, _HID)


# speedup vs baseline: 4.0266x; 1.6698x over previous
import functools

import jax
import jax.numpy as jnp
from jax import lax
from jax.experimental import pallas as pl
from jax.experimental.pallas import tpu as pltpu
from jax.experimental.pallas import tpu_sc as plsc

_BATCH = 4096
_SEQ = 200
_HID = 64
_NW = 32
_BT = _BATCH // _NW
_NPAIR = _SEQ // 2
_LANES = 16
_NV = _HID // _LANES
_EPS = 1e-12
_UNROLL = 2


def _hsum16(v):
    total = lax.cumsum(v, axis=0)
    dnums = lax.GatherDimensionNumbers(
        offset_dims=(), collapsed_slice_dims=(0,), start_index_map=(0,))
    idx = jnp.full((_LANES,), _LANES - 1, jnp.int32)
    return lax.gather(total, idx[:, None], dnums, slice_sizes=(1,),
                      mode=lax.GatherScatterMode.PROMISE_IN_BOUNDS)


def _rsqrt16(v):
    i = lax.bitcast_convert_type(v, jnp.int32)
    i = jnp.int32(0x5F3759DF) - lax.shift_right_arithmetic(i, jnp.int32(1))
    y = lax.bitcast_convert_type(i, jnp.float32)
    half = v * 0.5
    for _ in range(2):
        y = y * (1.5 - half * y * y)
    return y


def _sc_body(ids_hbm, table_hbm, pos_hbm, gam_hbm, bet_hbm, out_hbm,
             idx_v, rows0, rows1, outt0, outt1, pos_v, gam_v, bet_v,
             gsem, ssem_e, ssem_o):
    wid = lax.axis_index("s") * 2 + lax.axis_index("c")

    pltpu.sync_copy(pos_hbm, pos_v)
    pltpu.sync_copy(gam_hbm, gam_v)
    pltpu.sync_copy(bet_hbm, bet_v)
    pltpu.sync_copy(ids_hbm.at[:, pl.ds(wid * _BT, _BT)], idx_v)

    gv = [gam_v[pl.ds(c * _LANES, _LANES)] for c in range(_NV)]
    bv = [bet_v[pl.ds(c * _LANES, _LANES)] for c in range(_NV)]

    def issue_gather(s, buf):
        pltpu.make_async_copy(
            table_hbm.at[idx_v.at[s]], buf, gsem).start()

    def wait_gather(buf):
        pltpu.make_async_copy(
            table_hbm.at[pl.ds(0, _BT), :], buf, gsem).wait()

    def issue_store(buf, s, sem):
        pltpu.make_async_copy(buf.at[:, :, pl.ds(0, _BT)],
                              out_hbm.at[s, :, wid, :, :], sem).start()

    def wait_store(sem):
        pltpu.make_async_copy(
            table_hbm.at[pl.ds(0, _BT), :], rows0, sem).wait()

    lane_c = [lax.iota(jnp.int32, _LANES) + (c * _LANES) for c in range(_NV)]
    ct = [lax.shift_right_arithmetic(lc, jnp.int32(3)) for lc in lane_c]
    cr = [lax.bitwise_and(lc, jnp.int32(7)) for lc in lane_c]

    def compute(s, rows, outt):
        pv = [pos_v[s, pl.ds(c * _LANES, _LANES)] for c in range(_NV)]

        @plsc.parallel_loop(0, _BT, 1, unroll=_UNROLL)
        def _row(i):
            x = [rows[i, pl.ds(c * _LANES, _LANES)] + pv[c]
                 for c in range(_NV)]
            ssum = x[0] + x[1] + x[2] + x[3]
            q = x[0] * x[0] + x[1] * x[1] + x[2] * x[2] + x[3] * x[3]
            mu = _hsum16(ssum) * (1.0 / _HID)
            var = _hsum16(q) * (1.0 / _HID) - mu * mu
            rstd = _rsqrt16(var + _EPS)
            ms = mu * rstd
            bi = jnp.broadcast_to(i, (_LANES,)).astype(jnp.int32)
            for c in range(_NV):
                plsc.store_scatter(outt, [ct[c], cr[c], bi],
                                   (x[c] * rstd - ms) * gv[c] + bv[c])

    def pair(k, carry):
        a = 2 * k
        wait_gather(rows0)

        @pl.when(k >= 1)
        def _():
            wait_store(ssem_e)

        issue_gather(a + 1, rows1)
        compute(a, rows0, outt0)
        issue_store(outt0, a, ssem_e)

        wait_gather(rows1)

        @pl.when(k >= 1)
        def _():
            wait_store(ssem_o)

        @pl.when(k < _NPAIR - 1)
        def _():
            issue_gather(a + 2, rows0)

        compute(a + 1, rows1, outt1)
        issue_store(outt1, a + 1, ssem_o)
        return carry

    issue_gather(0, rows0)
    lax.fori_loop(0, _NPAIR, pair, 0)
    wait_store(ssem_e)
    wait_store(ssem_o)


_mesh = plsc.VectorSubcoreMesh(core_axis_name="c", subcore_axis_name="s")

_sc_kernel = functools.partial(
    pl.kernel,
    out_type=jax.ShapeDtypeStruct(
        (_SEQ, _HID // 8, _NW, 8, _BT), jnp.float32),
    mesh=_mesh,
    compiler_params=pltpu.CompilerParams(
        use_tc_tiling_on_sc=False, needs_layout_passes=False),
    scratch_types=[
        pltpu.VMEM((_SEQ, _BT), jnp.int32),
        pltpu.VMEM((_BT, _HID), jnp.float32),
        pltpu.VMEM((_BT, _HID), jnp.float32),
        pltpu.VMEM((_HID // 8, 8, _BT + 1), jnp.float32),
        pltpu.VMEM((_HID // 8, 8, _BT + 1), jnp.float32),
        pltpu.VMEM((_SEQ, _HID), jnp.float32),
        pltpu.VMEM((_HID,), jnp.float32),
        pltpu.VMEM((_HID,), jnp.float32),
        pltpu.SemaphoreType.DMA,
        pltpu.SemaphoreType.DMA,
        pltpu.SemaphoreType.DMA,
    ],
)(_sc_body)


@jax.jit
def kernel(input_ids, item_table, pos_table, ln_gamma, ln_beta):
    ids_t = input_ids.T.astype(jnp.int32)
    out5 = _sc_kernel(ids_t, item_table, pos_table, ln_gamma, ln_beta)
    return out5.transpose(2, 4, 0, 1, 3).reshape(_BATCH, _SEQ, _HID)


# speedup vs baseline: 4.0344x; 1.0019x over previous
import functools

import jax
import jax.numpy as jnp
from jax import lax
from jax.experimental import pallas as pl
from jax.experimental.pallas import tpu as pltpu
from jax.experimental.pallas import tpu_sc as plsc

_BATCH = 4096
_SEQ = 200
_HID = 64
_NW = 32
_BT = _BATCH // _NW
_NPAIR = _SEQ // 2
_LANES = 16
_NV = _HID // _LANES
_EPS = 1e-12
_UNROLL = 2


def _hsum16(v):
    total = lax.cumsum(v, axis=0)
    dnums = lax.GatherDimensionNumbers(
        offset_dims=(), collapsed_slice_dims=(0,), start_index_map=(0,))
    idx = jnp.full((_LANES,), _LANES - 1, jnp.int32)
    return lax.gather(total, idx[:, None], dnums, slice_sizes=(1,),
                      mode=lax.GatherScatterMode.PROMISE_IN_BOUNDS)


def _rsqrt16(v):
    i = lax.bitcast_convert_type(v, jnp.int32)
    i = jnp.int32(0x5F3759DF) - lax.shift_right_arithmetic(i, jnp.int32(1))
    y = lax.bitcast_convert_type(i, jnp.float32)
    half = v * 0.5
    for _ in range(2):
        y = y * (1.5 - half * y * y)
    return y


def _sc_body(ids_hbm, table_hbm, pos_hbm, gam_hbm, bet_hbm, out_hbm,
             idx_v, rows0, rows1, outt0, outt1, pos_v, gam_v, bet_v,
             gsem, ssem_e, ssem_o):
    wid = lax.axis_index("s") * 2 + lax.axis_index("c")

    pltpu.sync_copy(pos_hbm, pos_v)
    pltpu.sync_copy(gam_hbm, gam_v)
    pltpu.sync_copy(bet_hbm, bet_v)
    pltpu.sync_copy(ids_hbm.at[:, pl.ds(wid * _BT, _BT)], idx_v)

    gv = [gam_v[pl.ds(c * _LANES, _LANES)] for c in range(_NV)]
    bv = [bet_v[pl.ds(c * _LANES, _LANES)] for c in range(_NV)]

    def issue_gather(s, buf):
        pltpu.make_async_copy(
            table_hbm.at[idx_v.at[s]], buf, gsem).start()

    def wait_gather(buf):
        pltpu.make_async_copy(
            table_hbm.at[pl.ds(0, _BT), :], buf, gsem).wait()

    def issue_store(buf, s, sem):
        pltpu.make_async_copy(buf.at[:, :, pl.ds(0, _BT)],
                              out_hbm.at[s, :, wid, :, :], sem).start()

    def wait_store(sem):
        pltpu.make_async_copy(
            table_hbm.at[pl.ds(0, _BT), :], rows0, sem).wait()

    lane_c = [lax.iota(jnp.int32, _LANES) + (c * _LANES) for c in range(_NV)]
    ct = [lax.shift_right_arithmetic(lc, jnp.int32(3)) for lc in lane_c]
    cr = [lax.bitwise_and(lc, jnp.int32(7)) for lc in lane_c]

    def compute(s, rows, outt):
        pv = [pos_v[s, pl.ds(c * _LANES, _LANES)] for c in range(_NV)]

        @plsc.parallel_loop(0, _BT, 1, unroll=_UNROLL)
        def _row(i):
            x = [rows[i, pl.ds(c * _LANES, _LANES)] + pv[c]
                 for c in range(_NV)]
            ssum = x[0] + x[1] + x[2] + x[3]
            q = x[0] * x[0] + x[1] * x[1] + x[2] * x[2] + x[3] * x[3]
            mu = _hsum16(ssum) * (1.0 / _HID)
            var = _hsum16(q) * (1.0 / _HID) - mu * mu
            rstd = _rsqrt16(var + _EPS)
            ms = mu * rstd
            bi = jnp.broadcast_to(i, (_LANES,)).astype(jnp.int32)
            for c in range(_NV):
                plsc.store_scatter(outt, [ct[c], cr[c], bi],
                                   (x[c] * rstd - ms) * gv[c] + bv[c])

    def pair(k, carry):
        a = 2 * k
        issue_gather(a + 1, rows1)
        wait_gather(rows0)

        @pl.when(k >= 1)
        def _():
            wait_store(ssem_e)

        compute(a, rows0, outt0)
        issue_store(outt0, a, ssem_e)

        @pl.when(k < _NPAIR - 1)
        def _():
            issue_gather(a + 2, rows0)

        wait_gather(rows1)

        @pl.when(k >= 1)
        def _():
            wait_store(ssem_o)

        compute(a + 1, rows1, outt1)
        issue_store(outt1, a + 1, ssem_o)
        return carry

    issue_gather(0, rows0)
    lax.fori_loop(0, _NPAIR, pair, 0)
    wait_store(ssem_e)
    wait_store(ssem_o)


_mesh = plsc.VectorSubcoreMesh(core_axis_name="c", subcore_axis_name="s")

_sc_kernel = functools.partial(
    pl.kernel,
    out_type=jax.ShapeDtypeStruct(
        (_SEQ, _HID // 8, _NW, 8, _BT), jnp.float32),
    mesh=_mesh,
    compiler_params=pltpu.CompilerParams(
        use_tc_tiling_on_sc=False, needs_layout_passes=False),
    scratch_types=[
        pltpu.VMEM((_SEQ, _BT), jnp.int32),
        pltpu.VMEM((_BT, _HID), jnp.float32),
        pltpu.VMEM((_BT, _HID), jnp.float32),
        pltpu.VMEM((_HID // 8, 8, _BT + 1), jnp.float32),
        pltpu.VMEM((_HID // 8, 8, _BT + 1), jnp.float32),
        pltpu.VMEM((_SEQ, _HID), jnp.float32),
        pltpu.VMEM((_HID,), jnp.float32),
        pltpu.VMEM((_HID,), jnp.float32),
        pltpu.SemaphoreType.DMA,
        pltpu.SemaphoreType.DMA,
        pltpu.SemaphoreType.DMA,
    ],
)(_sc_body)


@jax.jit
def kernel(input_ids, item_table, pos_table, ln_gamma, ln_beta):
    ids_t = input_ids.T.astype(jnp.int32)
    out5 = _sc_kernel(ids_t, item_table, pos_table, ln_gamma, ln_beta)
    return out5.transpose(2, 4, 0, 1, 3).reshape(_BATCH, _SEQ, _HID)
